# baseline (speedup 1.0000x reference)
import jax
import jax.numpy as jnp
from jax.experimental import pallas as pl
from jax.experimental.pallas import tpu as pltpu


def _split_body(x_ref, o1_ref, o2_ref):
    nt = o1_ref.shape[1]
    o1_ref[...] = x_ref[:, :nt, :]
    o2_ref[...] = x_ref[:, nt:, :]


def kernel(x):
    B, C, H, W = x.shape
    HW = H * W
    nt = HW // 2
    y = jnp.transpose(x, (0, 2, 3, 1)).reshape(B, HW, C)
    o1, o2 = pl.pallas_call(
        _split_body,
        grid=(B // 4, 2),
        in_specs=[pl.BlockSpec((4, HW, C // 2), lambda i, j: (i, 0, j))],
        out_specs=[
            pl.BlockSpec((4, nt, C // 2), lambda i, j: (i, 0, j)),
            pl.BlockSpec((4, nt, C // 2), lambda i, j: (i, 0, j)),
        ],
        out_shape=[
            jax.ShapeDtypeStruct((B, nt, C), x.dtype),
            jax.ShapeDtypeStruct((B, nt, C), x.dtype),
        ],
    )(y)
    X1 = o1.reshape(B, H, nt // W, C).transpose(0, 3, 1, 2)
    X2 = o2.reshape(B, H, nt // W, C).transpose(0, 3, 1, 2)
    return (X1, X2)


# speedup vs baseline: 1.0495x; 1.0495x over previous
import jax
import jax.numpy as jnp
from jax.experimental import pallas as pl
from jax.experimental.pallas import tpu as pltpu


def _split_body(x_hbm, o1_hbm, o2_hbm, scr, in_sem, out_sem):
    B, HW, C = x_hbm.shape
    nt = HW // 2
    for b in range(B):
        pltpu.make_async_copy(x_hbm.at[b], scr.at[b], in_sem.at[b]).start()
    for b in range(B):
        pltpu.make_async_copy(x_hbm.at[b], scr.at[b], in_sem.at[b]).wait()
        pltpu.make_async_copy(scr.at[b, pl.ds(0, nt)], o1_hbm.at[b], out_sem.at[b, 0]).start()
        pltpu.make_async_copy(scr.at[b, pl.ds(nt, nt)], o2_hbm.at[b], out_sem.at[b, 1]).start()
    for b in range(B):
        pltpu.make_async_copy(scr.at[b, pl.ds(0, nt)], o1_hbm.at[b], out_sem.at[b, 0]).wait()
        pltpu.make_async_copy(scr.at[b, pl.ds(nt, nt)], o2_hbm.at[b], out_sem.at[b, 1]).wait()


def kernel(x):
    B, C, H, W = x.shape
    HW = H * W
    nt = HW // 2
    y = jnp.transpose(x, (0, 2, 3, 1)).reshape(B, HW, C)
    o1, o2 = pl.pallas_call(
        _split_body,
        in_specs=[pl.BlockSpec(memory_space=pl.ANY)],
        out_specs=[
            pl.BlockSpec(memory_space=pl.ANY),
            pl.BlockSpec(memory_space=pl.ANY),
        ],
        out_shape=[
            jax.ShapeDtypeStruct((B, nt, C), x.dtype),
            jax.ShapeDtypeStruct((B, nt, C), x.dtype),
        ],
        scratch_shapes=[
            pltpu.VMEM((B, HW, C), jnp.float32),
            pltpu.SemaphoreType.DMA((B,)),
            pltpu.SemaphoreType.DMA((B, 2)),
        ],
    )(y)
    X1 = o1.reshape(B, H, nt // W, C).transpose(0, 3, 1, 2)
    X2 = o2.reshape(B, H, nt // W, C).transpose(0, 3, 1, 2)
    return (X1, X2)


# speedup vs baseline: 1.0851x; 1.0339x over previous
import jax
import jax.numpy as jnp
from jax.experimental import pallas as pl
from jax.experimental.pallas import tpu as pltpu


def _split_body(x_ref, o1_ref, o2_ref):
    nt = o1_ref.shape[1]
    o1_ref[...] = x_ref[:, :nt, :]
    o2_ref[...] = x_ref[:, nt:, :]


def kernel(x):
    B, C, H, W = x.shape
    HW = H * W
    nt = HW // 2
    y = jnp.transpose(x, (0, 2, 3, 1)).reshape(B, HW, C)
    o1, o2 = pl.pallas_call(
        _split_body,
        grid=(B // 4,),
        in_specs=[pl.BlockSpec((4, HW, C), lambda i: (i, 0, 0))],
        out_specs=[
            pl.BlockSpec((4, nt, C), lambda i: (i, 0, 0)),
            pl.BlockSpec((4, nt, C), lambda i: (i, 0, 0)),
        ],
        out_shape=[
            jax.ShapeDtypeStruct((B, nt, C), x.dtype),
            jax.ShapeDtypeStruct((B, nt, C), x.dtype),
        ],
    )(y)
    X1 = o1.reshape(B, H, nt // W, C).transpose(0, 3, 1, 2)
    X2 = o2.reshape(B, H, nt // W, C).transpose(0, 3, 1, 2)
    return (X1, X2)
